# baseline (speedup 1.0000x reference)
import functools

import jax
import jax.numpy as jnp
from jax.experimental import pallas as pl
from jax.experimental.pallas import tpu as pltpu

N, D_IN, D_OUT, E = 4096, 2048, 2048, 8
BLOCK_N = 512


def _moe_block_kernel(x_ref, gw_ref, gb_ref, ew_ref, eb_ref, out_ref):
    e = pl.program_id(1)
    x = x_ref[...]

    gs = jax.lax.dot_general(
        x, gw_ref[...], (((1,), (1,)), ((), ())),
        preferred_element_type=jnp.float32,
    ) + gb_ref[...]

    lane = jax.lax.broadcasted_iota(jnp.int32, gs.shape, 1)
    m1 = jnp.max(gs, axis=1, keepdims=True)
    i1 = jnp.min(jnp.where(gs == m1, lane, E), axis=1, keepdims=True)
    masked = jnp.where(lane == i1, -jnp.inf, gs)
    m2 = jnp.max(masked, axis=1, keepdims=True)
    i2 = jnp.min(jnp.where(masked == m2, lane, E), axis=1, keepdims=True)

    w1 = 1.0 / (1.0 + jnp.exp(m2 - m1))
    w2 = 1.0 - w1
    we = w1 * (i1 == e) + w2 * (i2 == e)

    y = jax.lax.dot_general(
        x.astype(jnp.bfloat16), ew_ref[0].astype(jnp.bfloat16),
        (((1,), (1,)), ((), ())),
        preferred_element_type=jnp.float32,
    ) + eb_ref[0]

    @pl.when(e == 0)
    def _():
        out_ref[...] = jnp.zeros_like(out_ref)

    out_ref[...] += we * y


@jax.jit
def kernel(x, gate_W, gate_b, expert_W, expert_b):
    grid = (N // BLOCK_N, E)
    return pl.pallas_call(
        _moe_block_kernel,
        grid=grid,
        in_specs=[
            pl.BlockSpec((BLOCK_N, D_IN), lambda n, e: (n, 0)),
            pl.BlockSpec((E, D_IN), lambda n, e: (0, 0)),
            pl.BlockSpec((1, E), lambda n, e: (0, 0)),
            pl.BlockSpec((1, D_OUT, D_IN), lambda n, e: (e, 0, 0)),
            pl.BlockSpec((1, 1, D_OUT), lambda n, e: (e, 0, 0)),
        ],
        out_specs=pl.BlockSpec((BLOCK_N, D_OUT), lambda n, e: (n, 0)),
        out_shape=jax.ShapeDtypeStruct((N, D_OUT), jnp.float32),
        compiler_params=pltpu.CompilerParams(
            dimension_semantics=("parallel", "arbitrary"),
        ),
    )(x, gate_W, gate_b.reshape(1, E), expert_W, expert_b.reshape(E, 1, D_OUT))


# speedup vs baseline: 1.0239x; 1.0239x over previous
import functools

import jax
import jax.numpy as jnp
from jax.experimental import pallas as pl
from jax.experimental.pallas import tpu as pltpu

N, D_IN, D_OUT, E = 4096, 2048, 2048, 8
BLOCK_N = 512


def _moe_block_kernel(x_ref, gw_ref, gb_ref, ew_ref, eb_ref, out_ref, w8_ref):
    e = pl.program_id(1)
    x = x_ref[...]

    @pl.when(e == 0)
    def _():
        gs = jax.lax.dot_general(
            x, gw_ref[...], (((1,), (1,)), ((), ())),
            preferred_element_type=jnp.float32,
        ) + gb_ref[...]
        lane = jax.lax.broadcasted_iota(jnp.int32, gs.shape, 1)
        m1 = jnp.max(gs, axis=1, keepdims=True)
        i1 = jnp.min(jnp.where(gs == m1, lane, E), axis=1, keepdims=True)
        masked = jnp.where(lane == i1, -jnp.inf, gs)
        m2 = jnp.max(masked, axis=1, keepdims=True)
        i2 = jnp.min(jnp.where(masked == m2, lane, E), axis=1, keepdims=True)
        w1 = 1.0 / (1.0 + jnp.exp(m2 - m1))
        w8_ref[...] = jnp.where(
            lane == i1, w1, jnp.where(lane == i2, 1.0 - w1, 0.0)
        )
        out_ref[...] = jnp.zeros_like(out_ref)

    lane = jax.lax.broadcasted_iota(jnp.int32, (BLOCK_N, E), 1)
    we = jnp.sum(w8_ref[...] * (lane == e), axis=1, keepdims=True)

    y = jax.lax.dot_general(
        x, ew_ref[0], (((1,), (1,)), ((), ())),
        preferred_element_type=jnp.float32,
    ) + eb_ref[0]
    out_ref[...] += we * y


@jax.jit
def kernel(x, gate_W, gate_b, expert_W, expert_b):
    grid = (N // BLOCK_N, E)
    return pl.pallas_call(
        _moe_block_kernel,
        grid=grid,
        in_specs=[
            pl.BlockSpec((BLOCK_N, D_IN), lambda n, e: (n, 0)),
            pl.BlockSpec((E, D_IN), lambda n, e: (0, 0)),
            pl.BlockSpec((1, E), lambda n, e: (0, 0)),
            pl.BlockSpec((1, D_OUT, D_IN), lambda n, e: (e, 0, 0)),
            pl.BlockSpec((1, 1, D_OUT), lambda n, e: (e, 0, 0)),
        ],
        out_specs=pl.BlockSpec((BLOCK_N, D_OUT), lambda n, e: (n, 0)),
        out_shape=jax.ShapeDtypeStruct((N, D_OUT), jnp.float32),
        scratch_shapes=[pltpu.VMEM((BLOCK_N, E), jnp.float32)],
        compiler_params=pltpu.CompilerParams(
            dimension_semantics=("parallel", "arbitrary"),
        ),
    )(x, gate_W, gate_b.reshape(1, E), expert_W, expert_b.reshape(E, 1, D_OUT))
